# baseline (speedup 1.0000x reference)
import functools

import jax
import jax.numpy as jnp
from jax import lax
from jax.experimental import pallas as pl
from jax.experimental.pallas import tpu as pltpu
from jax.experimental.pallas import tpu_sc as plsc

B, S, F = 1024, 50, 26
NSPARSE = F - 1
EMB = 32
OUT_W = 1 + NSPARSE * EMB
S_PAD = 56
W_PAD = 896
H0 = 24
H1 = S - H0
R1 = S_PAD - H0

NC, NS = 2, 16
NW = NC * NS
TRIPS = B // NW
L = 16


def _sc_body(xi_hbm, xf_hbm, table_hbm, out_hbm,
             idsW, dvW, idxT, g0, g1, a0, a1,
             sg0, sg1, so0, so1):
    cid = lax.axis_index("c")
    sid = lax.axis_index("s")
    wid = sid * NC + cid
    iota = lax.iota(jnp.int32, L)
    zeros = jnp.zeros((L,), jnp.int32)
    blocks = ((0, 0, H0), (1, H0, H1))

    def assemble(gb, am, base, n):
        def pos(p, c2):
            for r in range(NSPARSE):
                row = r * n + p
                am[p, pl.ds(1 + r * EMB, L)] = gb[row, pl.ds(0, L)]
                am[p, pl.ds(1 + r * EMB + L, L)] = gb[row, pl.ds(L, L)]
            return c2

        lax.fori_loop(0, n, pos, 0)
        for off in (0, n - L):
            vals = plsc.load_gather(dvW, [iota + base + off, zeros])
            plsc.store_scatter(am, [iota + off, zeros], vals)

    def chunk(i, carry):
        b = i * NW + wid
        pltpu.sync_copy(xi_hbm.at[b], idsW)
        pltpu.sync_copy(xf_hbm.at[b], dvW)
        for j in range(NSPARSE):
            col = jnp.full((L,), j + 1, jnp.int32)
            for blk, base, n in blocks:
                for off in (0, n - L):
                    v = plsc.load_gather(idsW, [iota + base + off, col])
                    idxT[2 * j + blk, pl.ds(off, L)] = v
        gath0 = [
            pltpu.async_copy(table_hbm.at[idxT.at[2 * j, pl.ds(0, H0)]],
                             g0.at[pl.ds(j * H0, H0), :], sg0)
            for j in range(NSPARSE)
        ]
        gath1 = [
            pltpu.async_copy(table_hbm.at[idxT.at[2 * j + 1]],
                             g1.at[pl.ds(j * H1, H1), :], sg1)
            for j in range(NSPARSE)
        ]

        @pl.when(i > 0)
        def _():
            pltpu.make_async_copy(a0, out_hbm.at[b, pl.ds(0, H0), :],
                                  so0).wait()

        for cp in gath0:
            cp.wait()
        assemble(g0, a0, 0, H0)
        pltpu.async_copy(a0, out_hbm.at[b, pl.ds(0, H0), :], so0)

        @pl.when(i > 0)
        def _():
            pltpu.make_async_copy(a1, out_hbm.at[b, pl.ds(H0, R1), :],
                                  so1).wait()

        for cp in gath1:
            cp.wait()
        assemble(g1, a1, H0, H1)
        pltpu.async_copy(a1, out_hbm.at[b, pl.ds(H0, R1), :], so1)
        return carry

    lax.fori_loop(0, TRIPS, chunk, 0)
    pltpu.make_async_copy(a0, out_hbm.at[0, pl.ds(0, H0), :], so0).wait()
    pltpu.make_async_copy(a1, out_hbm.at[0, pl.ds(H0, R1), :], so1).wait()


def _tc_copy_body(src_ref, dst_ref):
    dst_ref[...] = src_ref[:, :S, :OUT_W]


BB = 8


@jax.jit
def _sc_call(xi, xf, table):
    padded = pl.kernel(
        _sc_body,
        out_type=jax.ShapeDtypeStruct((B, S_PAD, W_PAD), jnp.float32),
        mesh=plsc.VectorSubcoreMesh(
            core_axis_name="c", subcore_axis_name="s",
            num_cores=NC, num_subcores=NS,
        ),
        scratch_types=[
            pltpu.VMEM((S, F), jnp.int32),
            pltpu.VMEM((S, F), jnp.float32),
            pltpu.VMEM((2 * NSPARSE, H1), jnp.int32),
            pltpu.VMEM((H0 * NSPARSE, EMB), jnp.float32),
            pltpu.VMEM((H1 * NSPARSE, EMB), jnp.float32),
            pltpu.VMEM((H0, W_PAD), jnp.float32),
            pltpu.VMEM((R1, W_PAD), jnp.float32),
            pltpu.SemaphoreType.DMA,
            pltpu.SemaphoreType.DMA,
            pltpu.SemaphoreType.DMA,
            pltpu.SemaphoreType.DMA,
        ],
        compiler_params=pltpu.CompilerParams(
            use_tc_tiling_on_sc=False, needs_layout_passes=False),
    )(xi, xf, table)
    return pl.pallas_call(
        _tc_copy_body,
        grid=(B // BB,),
        in_specs=[pl.BlockSpec((BB, S_PAD, W_PAD), lambda i: (i, 0, 0))],
        out_specs=pl.BlockSpec((BB, S, OUT_W), lambda i: (i, 0, 0)),
        out_shape=jax.ShapeDtypeStruct((B, S, OUT_W), jnp.float32),
    )(padded)


def kernel(x, emb0):
    return _sc_call(x.astype(jnp.int32), x, emb0)
